# Initial kernel scaffold; baseline (speedup 1.0000x reference)
import jax
import jax.numpy as jnp
from jax.experimental import pallas as pl


def kernel(node_features, edge_features, senders, receivers, edge_W1, edge_b1, edge_W2, edge_b2, edge_g, edge_beta, node_W1, node_b1, node_W2, node_b2, node_g, node_beta, att_W, att_b):
    raise NotImplementedError("write your pallas kernel here")


# speedup vs baseline: 6.1311x; 6.1311x over previous
import jax
import jax.numpy as jnp
import numpy as np
from jax import lax
from jax.experimental import pallas as pl
from jax.experimental.pallas import tpu as pltpu
from jax.experimental.pallas import tpu_sc as plsc

N = 10000
E = 160000
D = 256
STEPS = 2

NC = 2
NS = 16
EH = E // 2
E_PER_SC = EH // NC
TILE_E = 2560
TILE_E_LAST = E_PER_SC - (NS - 1) * TILE_E
GCH = 80
SCH = 80
NCHS = TILE_E // SCH
DH = 128
EB = 1000
NB = 1000
NPAD = 10240
DEN_TILE = NPAD // NS
AGG_TILE = 624

_SC_MESH = plsc.VectorSubcoreMesh(
    core_axis_name="c", subcore_axis_name="s", num_cores=NC, num_subcores=NS)


def _proj_body(x_ref, wa_ref, wb_ref, ps_ref, pr_ref):
    x = x_ref[...]
    ps_ref[...] = jnp.dot(x, wa_ref[...], preferred_element_type=jnp.float32)
    pr_ref[...] = jnp.dot(x, wb_ref[...], preferred_element_type=jnp.float32)


def _project(nf, wa, wb):
    return pl.pallas_call(
        _proj_body,
        grid=(N // NB,),
        in_specs=[
            pl.BlockSpec((NB, D), lambda i: (i, 0)),
            pl.BlockSpec((D, D), lambda i: (0, 0)),
            pl.BlockSpec((D, D), lambda i: (0, 0)),
        ],
        out_specs=[pl.BlockSpec((NB, D), lambda i: (i, 0))] * 2,
        out_shape=[jax.ShapeDtypeStruct((N, D), jnp.float32)] * 2,
    )(nf, wa, wb)


def _edge_body(g_ref, ef_ref, w1_ref, w2_ref, p_ref, v_ref, efo_ref, ex_ref):
    x = ef_ref[...]
    z = g_ref[...] + jnp.dot(
        x.astype(jnp.bfloat16), w1_ref[...],
        preferred_element_type=jnp.float32) + p_ref[0]
    h = jnp.maximum(z, 0.0)
    y = jnp.dot(h.astype(jnp.bfloat16), w2_ref[...],
                preferred_element_type=jnp.float32) + p_ref[1]
    mu = jnp.mean(y, axis=1, keepdims=True)
    dev = y - mu
    var = jnp.mean(dev * dev, axis=1, keepdims=True)
    yn = dev * lax.rsqrt(var + 1e-5)
    ne = p_ref[2] * yn + p_ref[3]
    efo_ref[...] = x + ne
    lg = jnp.sum(ne * p_ref[4], axis=1, keepdims=True) + p_ref[5, 0]
    lg = jnp.where(lg > 0, lg, 0.2 * lg)
    ex = jnp.exp(jnp.minimum(lg, 30.0))
    ex_ref[...] = ex
    v_ref[...] = ne * ex


def _make_edge_mlp(half):
    boff = half * (EH // EB)

    def call(g, ef, w1c, w2, pe):
        return pl.pallas_call(
            _edge_body,
            grid=(EH // EB,),
            in_specs=[
                pl.BlockSpec((EB, D), lambda i: (i, 0)),
                pl.BlockSpec((EB, D), lambda i: (i + boff, 0)),
                pl.BlockSpec((D, D), lambda i: (0, 0)),
                pl.BlockSpec((D, D), lambda i: (0, 0)),
                pl.BlockSpec((8, D), lambda i: (0, 0)),
            ],
            out_specs=[
                pl.BlockSpec((EB, D), lambda i: (i, 0)),
                pl.BlockSpec((EB, D), lambda i: (i + boff, 0)),
                pl.BlockSpec((EB, 1), lambda i: (i, 0)),
            ],
            out_shape=[
                jax.ShapeDtypeStruct((EH, D), jnp.float32),
                jax.ShapeDtypeStruct((E, D), jnp.float32),
                jax.ShapeDtypeStruct((EH, 1), jnp.float32),
            ],
            input_output_aliases={1: 1},
        )(g, ef, w1c, w2, pe)

    return call


_EDGE_MLP = (_make_edge_mlp(0), _make_edge_mlp(1))


def _node_core(nf_ref, ap_ref, aq_ref, rden_ref, w1a_ref, w1b_ref, w2_ref,
               p_ref):
    agg = (ap_ref[0] + ap_ref[1] + aq_ref[0] + aq_ref[1]) * rden_ref[...]
    x = nf_ref[...]
    z = (jnp.dot(x, w1a_ref[...], preferred_element_type=jnp.float32)
         + jnp.dot(agg, w1b_ref[...], preferred_element_type=jnp.float32)
         + p_ref[0])
    h = jnp.maximum(z, 0.0)
    y = jnp.dot(h, w2_ref[...], preferred_element_type=jnp.float32) + p_ref[1]
    mu = jnp.mean(y, axis=1, keepdims=True)
    dev = y - mu
    var = jnp.mean(dev * dev, axis=1, keepdims=True)
    yn = dev * lax.rsqrt(var + 1e-5)
    return x + p_ref[2] * yn + p_ref[3]


def _node_body(nf_ref, ap_ref, aq_ref, rden_ref, w1a_ref, w1b_ref, w2_ref,
               p_ref, out_ref):
    out_ref[...] = _node_core(nf_ref, ap_ref, aq_ref, rden_ref, w1a_ref,
                              w1b_ref, w2_ref, p_ref)


def _node_proj_body(nf_ref, ap_ref, aq_ref, rden_ref, w1a_ref, w1b_ref,
                    w2_ref, p_ref, wea_ref, web_ref, out_ref, ps_ref,
                    pr_ref):
    nfo = _node_core(nf_ref, ap_ref, aq_ref, rden_ref, w1a_ref, w1b_ref,
                     w2_ref, p_ref)
    out_ref[...] = nfo
    ps_ref[...] = jnp.dot(nfo, wea_ref[...],
                          preferred_element_type=jnp.float32)
    pr_ref[...] = jnp.dot(nfo, web_ref[...],
                          preferred_element_type=jnp.float32)


_NODE_SPECS = [
    pl.BlockSpec((NB, D), lambda i: (i, 0)),
    pl.BlockSpec((NC, NB, D), lambda i: (0, i, 0)),
    pl.BlockSpec((NC, NB, D), lambda i: (0, i, 0)),
    pl.BlockSpec((NB, 1), lambda i: (i, 0)),
    pl.BlockSpec((D, D), lambda i: (0, 0)),
    pl.BlockSpec((D, D), lambda i: (0, 0)),
    pl.BlockSpec((D, D), lambda i: (0, 0)),
    pl.BlockSpec((8, D), lambda i: (0, 0)),
]


def _node_mlp(nf, agg_p, agg_q, rden, w1a, w1b, w2, pn):
    return pl.pallas_call(
        _node_body,
        grid=(N // NB,),
        in_specs=_NODE_SPECS,
        out_specs=pl.BlockSpec((NB, D), lambda i: (i, 0)),
        out_shape=jax.ShapeDtypeStruct((N, D), jnp.float32),
    )(nf, agg_p, agg_q, rden, w1a, w1b, w2, pn)


def _node_mlp_proj(nf, agg_p, agg_q, rden, w1a, w1b, w2, pn, wea, web):
    return pl.pallas_call(
        _node_proj_body,
        grid=(N // NB,),
        in_specs=_NODE_SPECS + [
            pl.BlockSpec((D, D), lambda i: (0, 0)),
            pl.BlockSpec((D, D), lambda i: (0, 0)),
        ],
        out_specs=[pl.BlockSpec((NB, D), lambda i: (i, 0))] * 3,
        out_shape=[jax.ShapeDtypeStruct((N, D), jnp.float32)] * 3,
    )(nf, agg_p, agg_q, rden, w1a, w1b, w2, pn, wea, web)


def _make_sc_gather(eoff):
    def body(ps_hbm, pr_hbm, snd_hbm, rcv_hbm, g_hbm,
             snd1d, rcv1d, a0, a1, b0, b1, sa0, sa1, sb0, sb1):
        c = lax.axis_index("c")
        s = lax.axis_index("s")
        base = c * E_PER_SC + s * TILE_E
        nch = jnp.where(s == NS - 1, TILE_E_LAST // GCH, TILE_E // GCH)
        npair = jnp.where(s == NS - 1, TILE_E_LAST // GCH // 2,
                          TILE_E // GCH // 2)

        @pl.when(s == NS - 1)
        def _():
            pltpu.sync_copy(snd_hbm.at[pl.ds(eoff + base, TILE_E_LAST)],
                            snd1d.at[pl.ds(0, TILE_E_LAST)])
            pltpu.sync_copy(rcv_hbm.at[pl.ds(eoff + base, TILE_E_LAST)],
                            rcv1d.at[pl.ds(0, TILE_E_LAST)])

        @pl.when(s < NS - 1)
        def _():
            pltpu.sync_copy(snd_hbm.at[pl.ds(eoff + base, TILE_E)], snd1d)
            pltpu.sync_copy(rcv_hbm.at[pl.ds(eoff + base, TILE_E)], rcv1d)

        bufs = ((a0, sa0, b0, sb0), (a1, sa1, b1, sb1))

        def g_copies(jj, slot):
            a, sa, b, sb = bufs[slot]
            isl = pl.ds(jj * GCH, GCH)
            return (pltpu.make_async_copy(ps_hbm.at[snd1d.at[isl]], a, sa),
                    pltpu.make_async_copy(pr_hbm.at[rcv1d.at[isl]], b, sb))

        for cp in g_copies(0, 0):
            cp.start()

        def pair(t, carry):
            for bslot in range(2):
                jj = 2 * t + bslot
                nxt = jj + 1

                @pl.when(nxt < nch)
                def _():
                    for cp in g_copies(nxt, (bslot + 1) % 2):
                        cp.start()

                for cp in g_copies(jj, bslot):
                    cp.wait()
                a = bufs[bslot][0]
                b = bufs[bslot][2]

                def row(r, carry2):
                    for k in range(D // 16):
                        sl = pl.ds(k * 16, 16)
                        a[r, sl] = a[r, sl] + b[r, sl]
                    return carry2

                lax.fori_loop(0, GCH, row, 0)
                pltpu.sync_copy(a, g_hbm.at[pl.ds(base + jj * GCH, GCH)])
            return carry

        lax.fori_loop(0, npair, pair, 0)

    def call(ps, pr, snd, rcv):
        return pl.kernel(
            body,
            out_type=jax.ShapeDtypeStruct((EH, D), jnp.float32),
            mesh=_SC_MESH,
            scratch_types=[
                pltpu.VMEM((TILE_E,), jnp.int32),
                pltpu.VMEM((TILE_E,), jnp.int32),
                pltpu.VMEM((GCH, D), jnp.float32),
                pltpu.VMEM((GCH, D), jnp.float32),
                pltpu.VMEM((GCH, D), jnp.float32),
                pltpu.VMEM((GCH, D), jnp.float32),
                pltpu.SemaphoreType.DMA,
                pltpu.SemaphoreType.DMA,
                pltpu.SemaphoreType.DMA,
                pltpu.SemaphoreType.DMA,
            ],
        )(ps, pr, snd, rcv)

    return call


_SC_GATHER = (_make_sc_gather(0), _make_sc_gather(EH))


def _make_sc_scatter(eoff):
    def body(ex_hbm, rcv_hbm, v_hbm, den_hbm, agg_hbm,
             rcv1d, ex1d, rcv2d, zden, vb0, vb1, zval, den_sh, agg_sh,
             sv0, sv1, ssc0, ssc1, sden):
        ssc = (ssc0, ssc1)
        c = lax.axis_index("c")
        s = lax.axis_index("s")
        base = c * E_PER_SC + s * TILE_E
        nch = jnp.where(s == NS - 1, TILE_E_LAST // SCH, NCHS)
        npair = jnp.where(s == NS - 1, TILE_E_LAST // SCH // 2, NCHS // 2)

        @pl.when(s == NS - 1)
        def _():
            pltpu.sync_copy(rcv_hbm.at[pl.ds(eoff + base, TILE_E_LAST)],
                            rcv1d.at[pl.ds(0, TILE_E_LAST)])
            pltpu.sync_copy(ex_hbm.at[pl.ds(base, TILE_E_LAST)],
                            ex1d.at[pl.ds(0, TILE_E_LAST)])

        @pl.when(s < NS - 1)
        def _():
            pltpu.sync_copy(rcv_hbm.at[pl.ds(eoff + base, TILE_E)], rcv1d)
            pltpu.sync_copy(ex_hbm.at[pl.ds(base, TILE_E)], ex1d)

        def fill2d(j, carry):
            for k in range(SCH // 16):
                rcv2d[j, pl.ds(k * 16, 16)] = (
                    rcv1d[pl.ds(j * SCH + k * 16, 16)])
            return carry

        lax.fori_loop(0, nch, fill2d, 0)

        zeros16 = jnp.zeros((16,), jnp.float32)
        for k in range(zden.shape[0] // 16):
            zden[pl.ds(k * 16, 16)] = zeros16

        def zrow(r, carry):
            for k in range(DH // 16):
                zval[r, pl.ds(k * 16, 16)] = zeros16
            return carry

        lax.fori_loop(0, zval.shape[0], zrow, 0)

        pltpu.sync_copy(zden, den_sh.at[pl.ds(s * DEN_TILE, DEN_TILE)])

        vbufs = ((vb0, sv0), (vb1, sv1))

        for dh in range(D // DH):
            def v_copy(jj, slot, dh=dh):
                vb, sv = vbufs[slot]
                return pltpu.make_async_copy(
                    v_hbm.at[pl.ds(base + jj * SCH, SCH),
                             pl.ds(dh * DH, DH)],
                    vb, sv)

            nz = AGG_TILE // zval.shape[0]
            for k in range(nz):
                pltpu.sync_copy(
                    zval, agg_sh.at[pl.ds(s * AGG_TILE + k * zval.shape[0],
                                          zval.shape[0])])
            rem = AGG_TILE - nz * zval.shape[0]
            if rem:
                pltpu.sync_copy(
                    zval.at[pl.ds(0, rem)],
                    agg_sh.at[pl.ds(s * AGG_TILE + nz * zval.shape[0], rem)])

            @pl.when(s == NS - 1)
            def _():
                pltpu.sync_copy(zval.at[pl.ds(0, 16)],
                                agg_sh.at[pl.ds(NS * AGG_TILE, 16)])

            plsc.subcore_barrier()

            v_copy(0, 0).start()

            def den_copy(jj):
                return pltpu.make_async_copy(
                    ex1d.at[pl.ds(jj * SCH, SCH)],
                    den_sh.at[rcv2d.at[jj]], sden)

            if dh == 0:
                def den_fire(jj, carry):
                    pltpu.async_copy(ex1d.at[pl.ds(jj * SCH, SCH)],
                                     den_sh.at[rcv2d.at[jj]], sden,
                                     add=True)
                    return carry

                lax.fori_loop(0, nch, den_fire, 0)

            def sc_copy(jj, slot):
                return pltpu.make_async_copy(
                    vbufs[slot][0], agg_sh.at[rcv2d.at[jj]], ssc[slot])

            def pair(t, carry, v_copy=v_copy):
                for bslot in range(2):
                    jj = 2 * t + bslot
                    nxt = jj + 1
                    oslot = (bslot + 1) % 2

                    @pl.when(jj >= 1)
                    def _():
                        sc_copy(jj - 1, oslot).wait()

                    @pl.when(nxt < nch)
                    def _():
                        v_copy(nxt, oslot).start()

                    v_copy(jj, bslot).wait()
                    pltpu.async_copy(vbufs[bslot][0],
                                     agg_sh.at[rcv2d.at[jj]], ssc[bslot],
                                     add=True)
                return carry

            lax.fori_loop(0, npair, pair, 0)
            sc_copy(nch - 1, 1).wait()

            if dh == 0:
                def den_drain(jj, carry):
                    den_copy(jj).wait()
                    return carry

                lax.fori_loop(0, nch, den_drain, 0)
            plsc.subcore_barrier()

            pltpu.sync_copy(
                agg_sh.at[pl.ds(s * AGG_TILE, AGG_TILE)],
                agg_hbm.at[c, pl.ds(s * AGG_TILE, AGG_TILE),
                           pl.ds(dh * DH, DH)])

            @pl.when(s == NS - 1)
            def _():
                pltpu.sync_copy(
                    agg_sh.at[pl.ds(NS * AGG_TILE, 16)],
                    agg_hbm.at[c, pl.ds(NS * AGG_TILE, 16),
                               pl.ds(dh * DH, DH)])

            if dh == 0:
                pltpu.sync_copy(den_sh.at[pl.ds(s * DEN_TILE, DEN_TILE)],
                                den_hbm.at[c, pl.ds(s * DEN_TILE, DEN_TILE)])

    def call(ex, rcv, v):
        return pl.kernel(
            body,
            out_type=[jax.ShapeDtypeStruct((NC, NPAD), jnp.float32),
                      jax.ShapeDtypeStruct((NC, N, D), jnp.float32)],
            mesh=_SC_MESH,
            scratch_types=[
                pltpu.VMEM((TILE_E,), jnp.int32),
                pltpu.VMEM((TILE_E,), jnp.float32),
                pltpu.VMEM((NCHS, SCH), jnp.int32),
                pltpu.VMEM((DEN_TILE,), jnp.float32),
                pltpu.VMEM((SCH, DH), jnp.float32),
                pltpu.VMEM((SCH, DH), jnp.float32),
                pltpu.VMEM((64, DH), jnp.float32),
                pltpu.VMEM_SHARED((NPAD,), jnp.float32),
                pltpu.VMEM_SHARED((N, DH), jnp.float32),
                pltpu.SemaphoreType.DMA,
                pltpu.SemaphoreType.DMA,
                pltpu.SemaphoreType.DMA,
                pltpu.SemaphoreType.DMA,
                pltpu.SemaphoreType.DMA,
            ],
        )(ex, rcv, v)

    return call


_SC_SCATTER = (_make_sc_scatter(0), _make_sc_scatter(EH))


def kernel(node_features, edge_features, senders, receivers,
           edge_W1, edge_b1, edge_W2, edge_b2, edge_g, edge_beta,
           node_W1, node_b1, node_W2, node_b2, node_g, node_beta,
           att_W, att_b):
    nf = node_features[0]
    ef = edge_features[0]
    snd = senders.astype(jnp.int32)
    rcv = receivers.astype(jnp.int32)
    zrow2 = jnp.zeros((2, D), jnp.float32)
    ps, pr = _project(nf, edge_W1[0, :D], edge_W1[0, D:2 * D])
    for s in range(STEPS):
        W1 = edge_W1[s]
        w1c = W1[2 * D:].astype(jnp.bfloat16)
        w2e = edge_W2[s].astype(jnp.bfloat16)
        pe = jnp.concatenate([
            edge_b1[s][None], edge_b2[s][None],
            edge_g[s][None], edge_beta[s][None],
            att_W[s].T, jnp.full((1, D), att_b[s, 0]), zrow2], axis=0)
        g0 = _SC_GATHER[0](ps, pr, snd, rcv)
        g1 = _SC_GATHER[1](ps, pr, snd, rcv)
        v0, ef, ex0 = _EDGE_MLP[0](g0, ef, w1c, w2e, pe)
        v1, ef, ex1 = _EDGE_MLP[1](g1, ef, w1c, w2e, pe)
        d0, a0 = _SC_SCATTER[0](ex0.reshape(EH), rcv, v0)
        d1, a1 = _SC_SCATTER[1](ex1.reshape(EH), rcv, v1)
        rden = (1.0 / (d0[0, :N] + d0[1, :N] + d1[0, :N] + d1[1, :N]
                       + 1e-16)).reshape(N, 1)
        pn = jnp.concatenate([
            node_b1[s][None], node_b2[s][None],
            node_g[s][None], node_beta[s][None],
            zrow2, zrow2], axis=0)
        if s < STEPS - 1:
            nf, ps, pr = _node_mlp_proj(
                nf, a0, a1, rden, node_W1[s][:D], node_W1[s][D:],
                node_W2[s], pn,
                edge_W1[s + 1, :D], edge_W1[s + 1, D:2 * D])
        else:
            nf = _node_mlp(nf, a0, a1, rden,
                           node_W1[s][:D], node_W1[s][D:], node_W2[s], pn)
    return nf[None], ef[None]
